# baseline (speedup 1.0000x reference)
import functools

import jax
import jax.numpy as jnp
from jax import lax
from jax.experimental import pallas as pl
from jax.experimental.pallas import tpu as pltpu
from jax.experimental.pallas import tpu_sc as plsc

NC = 2
NS = 16
L = 16
EC = 200


def _ceil(a, b):
    return -(-a // b)


def _sc_degree(ei, NP, EPW):
    stripe = NP // NS
    KF = EPW // EC
    TR = EPW - KF * EC
    mesh = plsc.VectorSubcoreMesh(core_axis_name="c", subcore_axis_name="s")

    @functools.partial(
        pl.kernel,
        out_type=jax.ShapeDtypeStruct((NC, NP, L), jnp.float32),
        mesh=mesh,
        compiler_params=pltpu.CompilerParams(use_tc_tiling_on_sc=False),
        scratch_types=[
            pltpu.VMEM((EPW,), jnp.int32),
            pltpu.VMEM((EC, L), jnp.float32),
            pltpu.VMEM((EC, L), jnp.float32),
            pltpu.VMEM_SHARED((NP, L), jnp.float32),
            pltpu.SemaphoreType.DMA,
        ],
    )
    def k(ei_hbm, out_hbm, dst_v, zero_v, one_v, acc, dsem):
        c = lax.axis_index("c")
        s = lax.axis_index("s")
        w = c * NS + s

        def fill(i, _):
            zero_v[i, :] = jnp.zeros((L,), jnp.float32)
            one_v[i, :] = jnp.ones((L,), jnp.float32)
            return _

        lax.fori_loop(0, EC, fill, 0)
        tb = s * stripe
        for b in range(stripe // EC):
            pltpu.sync_copy(zero_v, acc.at[pl.ds(tb + b * EC, EC)])
        rem = stripe - (stripe // EC) * EC
        if rem:
            pltpu.sync_copy(zero_v.at[pl.ds(0, rem)],
                            acc.at[pl.ds(tb + (stripe // EC) * EC, rem)])
        pltpu.sync_copy(ei_hbm.at[1, pl.ds(w * EPW, EPW)], dst_v)
        plsc.subcore_barrier()

        def chunk(j, _):
            pltpu.async_copy(one_v, acc.at[dst_v.at[pl.ds(j * EC, EC)]],
                             dsem, add=True)
            return _

        lax.fori_loop(0, KF, chunk, 0)
        if TR:
            pltpu.sync_copy(one_v.at[pl.ds(0, TR)],
                            acc.at[dst_v.at[pl.ds(KF * EC, TR)]], add=True)

        def drain(j, _):
            pltpu.make_async_copy(one_v, acc.at[dst_v.at[pl.ds(j * EC, EC)]],
                                  dsem).wait()
            return _

        lax.fori_loop(0, KF, drain, 0)
        plsc.subcore_barrier()
        pltpu.sync_copy(acc.at[pl.ds(tb, stripe)],
                        out_hbm.at[c, pl.ds(tb, stripe)])

    return k(ei)


def _sc_propagate(table, ei, EPW):
    NP, D = table.shape
    stripe = NP // NS
    KF = EPW // EC
    TR = EPW - KF * EC
    KP = KF // 2
    mesh = plsc.VectorSubcoreMesh(core_axis_name="c", subcore_axis_name="s")

    @functools.partial(
        pl.kernel,
        out_type=jax.ShapeDtypeStruct((NC, NP, D), jnp.float32),
        mesh=mesh,
        compiler_params=pltpu.CompilerParams(use_tc_tiling_on_sc=False),
        scratch_types=[
            pltpu.VMEM((EPW,), jnp.int32),
            pltpu.VMEM((EPW,), jnp.int32),
            pltpu.VMEM((EC, D), jnp.float32),
            pltpu.VMEM((EC, D), jnp.float32),
            pltpu.VMEM_SHARED((NP, D), jnp.float32),
            pltpu.SemaphoreType.DMA,
            pltpu.SemaphoreType.DMA,
            pltpu.SemaphoreType.DMA,
            pltpu.SemaphoreType.DMA,
        ],
    )
    def k(table_hbm, ei_hbm, out_hbm, src_v, dst_v, r0, r1,
          acc, g0, g1, s0, s1):
        rows = [r0, r1]
        gsem = [g0, g1]
        ssem = [s0, s1]
        c = lax.axis_index("c")
        s = lax.axis_index("s")
        w = c * NS + s

        def zrow(i, _):
            for t in range(D // L):
                r0[i, pl.ds(t * L, L)] = jnp.zeros((L,), jnp.float32)
            return _

        lax.fori_loop(0, EC, zrow, 0)
        tb = s * stripe
        for b in range(stripe // EC):
            pltpu.sync_copy(r0, acc.at[pl.ds(tb + b * EC, EC)])
        rem = stripe - (stripe // EC) * EC
        if rem:
            pltpu.sync_copy(r0.at[pl.ds(0, rem)],
                            acc.at[pl.ds(tb + (stripe // EC) * EC, rem)])
        pltpu.sync_copy(ei_hbm.at[0, pl.ds(w * EPW, EPW)], src_v)
        pltpu.sync_copy(ei_hbm.at[1, pl.ds(w * EPW, EPW)], dst_v)
        plsc.subcore_barrier()

        def sidx(j):
            return src_v.at[pl.ds(j * EC, EC)]

        def didx(j):
            return dst_v.at[pl.ds(j * EC, EC)]

        def gwait(slot, j):
            pltpu.make_async_copy(table_hbm.at[sidx(j)], rows[slot],
                                  gsem[slot]).wait()

        def swait(slot, j):
            pltpu.make_async_copy(rows[slot], acc.at[didx(j)],
                                  ssem[slot]).wait()

        if KP:
            pltpu.async_copy(table_hbm.at[sidx(0)], rows[0], gsem[0])
            pltpu.async_copy(table_hbm.at[sidx(1)], rows[1], gsem[1])

            def chunk2(jj, _):
                j0 = 2 * jj
                for i in range(2):
                    j = j0 + i
                    gwait(i, j)
                    pltpu.async_copy(rows[i], acc.at[didx(j)],
                                     ssem[i], add=True)
                for i in range(2):
                    j = j0 + i
                    swait(i, j)

                    @pl.when(jj < KP - 1)
                    def _ig():
                        pltpu.async_copy(table_hbm.at[sidx(j + 2)],
                                         rows[i], gsem[i])
                return _

            lax.fori_loop(0, KP, chunk2, 0)

        extras = []
        if KF % 2:
            extras.append((2 * KP * EC, EC))
        if TR:
            extras.append((KF * EC, TR))
        for off, sz in extras:
            si = src_v.at[pl.ds(off, sz)]
            di = dst_v.at[pl.ds(off, sz)]
            rs = r0.at[pl.ds(0, sz)]
            pltpu.async_copy(table_hbm.at[si], rs, g0).wait()
            pltpu.sync_copy(rs, acc.at[di], add=True)
        plsc.subcore_barrier()
        pltpu.sync_copy(acc.at[pl.ds(tb, stripe)],
                        out_hbm.at[c, pl.ds(tb, stripe)])

    return k(table, ei)


def _tc_matmul(xp, W1p):
    NP, D = xp.shape
    Hp = W1p.shape[1]
    RB = NP // 8

    def body(x_ref, w_ref, o_ref):
        o_ref[...] = jnp.dot(x_ref[...], w_ref[...],
                             preferred_element_type=jnp.float32)

    return pl.pallas_call(
        body,
        grid=(8,),
        in_specs=[pl.BlockSpec((RB, D), lambda i: (i, 0)),
                  pl.BlockSpec((D, Hp), lambda i: (0, 0))],
        out_specs=pl.BlockSpec((RB, Hp), lambda i: (i, 0)),
        out_shape=jax.ShapeDtypeStruct((NP, Hp), jnp.float32),
    )(xp, W1p)


def _tc_scale(h1, degp):
    NP, Hp = h1.shape
    RB = NP // 8

    def body(deg_ref, h1_ref, h1s_ref, dinv_ref):
        deg = deg_ref[0] + deg_ref[1] + 1.0
        dinv = lax.rsqrt(jnp.maximum(deg, 1.0))
        dinv_ref[...] = dinv
        h1s_ref[...] = h1_ref[...] * dinv[:, 0:1]

    return pl.pallas_call(
        body,
        grid=(8,),
        in_specs=[pl.BlockSpec((NC, RB, L), lambda i: (0, i, 0)),
                  pl.BlockSpec((RB, Hp), lambda i: (i, 0))],
        out_specs=[pl.BlockSpec((RB, Hp), lambda i: (i, 0)),
                   pl.BlockSpec((RB, L), lambda i: (i, 0))],
        out_shape=[jax.ShapeDtypeStruct((NP, Hp), jnp.float32),
                   jax.ShapeDtypeStruct((NP, L), jnp.float32)],
    )(degp, h1)


def _tc_layer2(p, h1s, dinv, b1p, W2p):
    NP, Hp = h1s.shape
    Cp = W2p.shape[1]
    RB = NP // 8

    def body(p_ref, h1s_ref, dinv_ref, b1_ref, w2_ref, gs_ref):
        d = dinv_ref[:, 0:1]
        h = jnp.maximum((p_ref[0] + p_ref[1] + h1s_ref[...]) * d + b1_ref[...],
                        0.0)
        gs_ref[...] = jnp.dot(h, w2_ref[...],
                              preferred_element_type=jnp.float32) * d

    return pl.pallas_call(
        body,
        grid=(8,),
        in_specs=[pl.BlockSpec((NC, RB, Hp), lambda i: (0, i, 0)),
                  pl.BlockSpec((RB, Hp), lambda i: (i, 0)),
                  pl.BlockSpec((RB, L), lambda i: (i, 0)),
                  pl.BlockSpec((1, Hp), lambda i: (0, 0)),
                  pl.BlockSpec((Hp, Cp), lambda i: (0, 0))],
        out_specs=pl.BlockSpec((RB, Cp), lambda i: (i, 0)),
        out_shape=jax.ShapeDtypeStruct((NP, Cp), jnp.float32),
    )(p, h1s, dinv, b1p, W2p)


def _tc_final(q, gs, dinv, b2p, Wcp, bcp, N, C):
    NP, Cp = gs.shape
    RB = None
    for g in (5, 4, 8, 10, 16, 20, 25):
        if N % g == 0 and (N // g) % 8 == 0:
            RB = N // g
            ngrid = g
            break

    def body(q_ref, gs_ref, dinv_ref, b2_ref, wc_ref, bc_ref, h2_ref, out_ref):
        d = dinv_ref[:, 0:1]
        h2 = (q_ref[0] + q_ref[1] + gs_ref[...]) * d + b2_ref[...]
        outp = jnp.dot(h2, wc_ref[...],
                       preferred_element_type=jnp.float32) + bc_ref[...]
        if RB is not None:
            h2_ref[...] = h2[:, :C]
            out_ref[...] = outp[:, :C]
        else:
            h2_ref[...] = h2
            out_ref[...] = outp

    if RB is None:
        RB = NP // 8
        h2p, outp = pl.pallas_call(
            body,
            grid=(8,),
            in_specs=[pl.BlockSpec((NC, RB, Cp), lambda i: (0, i, 0)),
                      pl.BlockSpec((RB, Cp), lambda i: (i, 0)),
                      pl.BlockSpec((RB, L), lambda i: (i, 0)),
                      pl.BlockSpec((1, Cp), lambda i: (0, 0)),
                      pl.BlockSpec((Cp, Cp), lambda i: (0, 0)),
                      pl.BlockSpec((1, Cp), lambda i: (0, 0))],
            out_specs=[pl.BlockSpec((RB, Cp), lambda i: (i, 0)),
                       pl.BlockSpec((RB, Cp), lambda i: (i, 0))],
            out_shape=[jax.ShapeDtypeStruct((NP, Cp), jnp.float32),
                       jax.ShapeDtypeStruct((NP, Cp), jnp.float32)],
        )(q, gs, dinv, b2p, Wcp, bcp)
        return h2p[:N, :C], outp[:N, :C]

    h2, out = pl.pallas_call(
        body,
        grid=(ngrid,),
        in_specs=[pl.BlockSpec((NC, RB, Cp), lambda i: (0, i, 0)),
                  pl.BlockSpec((RB, Cp), lambda i: (i, 0)),
                  pl.BlockSpec((RB, L), lambda i: (i, 0)),
                  pl.BlockSpec((1, Cp), lambda i: (0, 0)),
                  pl.BlockSpec((Cp, Cp), lambda i: (0, 0)),
                  pl.BlockSpec((1, Cp), lambda i: (0, 0))],
        out_specs=[pl.BlockSpec((RB, C), lambda i: (i, 0)),
                   pl.BlockSpec((RB, C), lambda i: (i, 0))],
        out_shape=[jax.ShapeDtypeStruct((N, C), jnp.float32),
                   jax.ShapeDtypeStruct((N, C), jnp.float32)],
    )(q, gs, dinv, b2p, Wcp, bcp)
    return h2, out


def kernel(x, edge_index, W1, b1, W2, b2, Wc, bc):
    N, D = x.shape
    H = W1.shape[1]
    C = W2.shape[1]
    E = edge_index.shape[1]
    NW = NC * NS
    NP = _ceil(N + 1, 128) * 128
    Hp = _ceil(H, L) * L
    Cp = L

    NW8 = NW * 8
    Epad = _ceil(E, NW8) * NW8
    ei = edge_index.astype(jnp.int32)
    if Epad != E:
        padv = N + (jnp.arange(Epad - E, dtype=jnp.int32) % (NP - N))
        ei = jnp.concatenate([ei, jnp.stack([padv, padv])], axis=1)
    EPW = Epad // NW

    xp = jnp.pad(x, ((0, NP - N), (0, 0)))
    W1p = jnp.pad(W1, ((0, 0), (0, Hp - H)))
    b1p = jnp.pad(b1, (0, Hp - H))[None, :]
    W2p = jnp.pad(W2, ((0, Hp - H), (0, Cp - C)))
    b2p = jnp.pad(b2, (0, Cp - C))[None, :]
    Wcp = jnp.pad(Wc, ((0, Cp - C), (0, Cp - C)))
    bcp = jnp.pad(bc, (0, Cp - C))[None, :]

    degp = _sc_degree(ei, NP, EPW)
    h1 = _tc_matmul(xp, W1p)
    h1s, dinv = _tc_scale(h1, degp)
    p = _sc_propagate(h1s, ei, EPW)
    gs = _tc_layer2(p, h1s, dinv, b1p, W2p)
    q = _sc_propagate(gs, ei, EPW)
    h2, out = _tc_final(q, gs, dinv, b2p, Wcp, bcp, N, C)
    return (out, h2)


# speedup vs baseline: 1.0102x; 1.0102x over previous
import functools

import jax
import jax.numpy as jnp
from jax import lax
from jax.experimental import pallas as pl
from jax.experimental.pallas import tpu as pltpu
from jax.experimental.pallas import tpu_sc as plsc

NC = 2
NS = 16
L = 16
EC = 256


def _ceil(a, b):
    return -(-a // b)


def _sc_degree(ei, NP, EPW):
    stripe = NP // NS
    KF = EPW // EC
    TR = EPW - KF * EC
    mesh = plsc.VectorSubcoreMesh(core_axis_name="c", subcore_axis_name="s")

    @functools.partial(
        pl.kernel,
        out_type=jax.ShapeDtypeStruct((NC, NP, L), jnp.float32),
        mesh=mesh,
        compiler_params=pltpu.CompilerParams(use_tc_tiling_on_sc=False),
        scratch_types=[
            pltpu.VMEM((EPW,), jnp.int32),
            pltpu.VMEM((EC, L), jnp.float32),
            pltpu.VMEM((EC, L), jnp.float32),
            pltpu.VMEM_SHARED((NP, L), jnp.float32),
            pltpu.SemaphoreType.DMA,
        ],
    )
    def k(ei_hbm, out_hbm, dst_v, zero_v, one_v, acc, dsem):
        c = lax.axis_index("c")
        s = lax.axis_index("s")
        w = c * NS + s

        def fill(i, _):
            zero_v[i, :] = jnp.zeros((L,), jnp.float32)
            one_v[i, :] = jnp.ones((L,), jnp.float32)
            return _

        lax.fori_loop(0, EC, fill, 0)
        tb = s * stripe
        for b in range(stripe // EC):
            pltpu.sync_copy(zero_v, acc.at[pl.ds(tb + b * EC, EC)])
        rem = stripe - (stripe // EC) * EC
        if rem:
            pltpu.sync_copy(zero_v.at[pl.ds(0, rem)],
                            acc.at[pl.ds(tb + (stripe // EC) * EC, rem)])
        pltpu.sync_copy(ei_hbm.at[1, pl.ds(w * EPW, EPW)], dst_v)
        plsc.subcore_barrier()

        def chunk(j, _):
            pltpu.async_copy(one_v, acc.at[dst_v.at[pl.ds(j * EC, EC)]],
                             dsem, add=True)
            return _

        lax.fori_loop(0, KF, chunk, 0)
        if TR:
            pltpu.sync_copy(one_v.at[pl.ds(0, TR)],
                            acc.at[dst_v.at[pl.ds(KF * EC, TR)]], add=True)

        def drain(j, _):
            pltpu.make_async_copy(one_v, acc.at[dst_v.at[pl.ds(j * EC, EC)]],
                                  dsem).wait()
            return _

        lax.fori_loop(0, KF, drain, 0)
        plsc.subcore_barrier()
        pltpu.sync_copy(acc.at[pl.ds(tb, stripe)],
                        out_hbm.at[c, pl.ds(tb, stripe)])

    return k(ei)


def _sc_propagate(table, ei, EPW):
    NP, D = table.shape
    stripe = NP // NS
    KF = EPW // EC
    TR = EPW - KF * EC
    KP = KF // 2
    mesh = plsc.VectorSubcoreMesh(core_axis_name="c", subcore_axis_name="s")

    @functools.partial(
        pl.kernel,
        out_type=jax.ShapeDtypeStruct((NC, NP, D), jnp.float32),
        mesh=mesh,
        compiler_params=pltpu.CompilerParams(use_tc_tiling_on_sc=False),
        scratch_types=[
            pltpu.VMEM((EPW,), jnp.int32),
            pltpu.VMEM((EPW,), jnp.int32),
            pltpu.VMEM((EC, D), jnp.float32),
            pltpu.VMEM((EC, D), jnp.float32),
            pltpu.VMEM_SHARED((NP, D), jnp.float32),
            pltpu.SemaphoreType.DMA,
            pltpu.SemaphoreType.DMA,
            pltpu.SemaphoreType.DMA,
            pltpu.SemaphoreType.DMA,
        ],
    )
    def k(table_hbm, ei_hbm, out_hbm, src_v, dst_v, r0, r1,
          acc, g0, g1, s0, s1):
        rows = [r0, r1]
        gsem = [g0, g1]
        ssem = [s0, s1]
        c = lax.axis_index("c")
        s = lax.axis_index("s")
        w = c * NS + s

        def zrow(i, _):
            for t in range(D // L):
                r0[i, pl.ds(t * L, L)] = jnp.zeros((L,), jnp.float32)
            return _

        lax.fori_loop(0, EC, zrow, 0)
        tb = s * stripe
        for b in range(stripe // EC):
            pltpu.sync_copy(r0, acc.at[pl.ds(tb + b * EC, EC)])
        rem = stripe - (stripe // EC) * EC
        if rem:
            pltpu.sync_copy(r0.at[pl.ds(0, rem)],
                            acc.at[pl.ds(tb + (stripe // EC) * EC, rem)])
        pltpu.sync_copy(ei_hbm.at[0, pl.ds(w * EPW, EPW)], src_v)
        pltpu.sync_copy(ei_hbm.at[1, pl.ds(w * EPW, EPW)], dst_v)
        plsc.subcore_barrier()

        def sidx(j):
            return src_v.at[pl.ds(j * EC, EC)]

        def didx(j):
            return dst_v.at[pl.ds(j * EC, EC)]

        def gwait(slot, j):
            pltpu.make_async_copy(table_hbm.at[sidx(j)], rows[slot],
                                  gsem[slot]).wait()

        def swait(slot, j):
            pltpu.make_async_copy(rows[slot], acc.at[didx(j)],
                                  ssem[slot]).wait()

        if KP:
            pltpu.async_copy(table_hbm.at[sidx(0)], rows[0], gsem[0])
            pltpu.async_copy(table_hbm.at[sidx(1)], rows[1], gsem[1])

            def chunk2(jj, _):
                j0 = 2 * jj
                for i in range(2):
                    j = j0 + i
                    gwait(i, j)
                    pltpu.async_copy(rows[i], acc.at[didx(j)],
                                     ssem[i], add=True)
                for i in range(2):
                    j = j0 + i
                    swait(i, j)

                    @pl.when(jj < KP - 1)
                    def _ig():
                        pltpu.async_copy(table_hbm.at[sidx(j + 2)],
                                         rows[i], gsem[i])
                return _

            lax.fori_loop(0, KP, chunk2, 0)

        extras = []
        if KF % 2:
            extras.append((2 * KP * EC, EC))
        if TR:
            extras.append((KF * EC, TR))
        for off, sz in extras:
            si = src_v.at[pl.ds(off, sz)]
            di = dst_v.at[pl.ds(off, sz)]
            rs = r0.at[pl.ds(0, sz)]
            pltpu.async_copy(table_hbm.at[si], rs, g0).wait()
            pltpu.sync_copy(rs, acc.at[di], add=True)
        plsc.subcore_barrier()
        pltpu.sync_copy(acc.at[pl.ds(tb, stripe)],
                        out_hbm.at[c, pl.ds(tb, stripe)])

    return k(table, ei)


def _tc_matmul(xp, W1p):
    NP, D = xp.shape
    Hp = W1p.shape[1]
    RB = NP // 8

    def body(x_ref, w_ref, o_ref):
        o_ref[...] = jnp.dot(x_ref[...], w_ref[...],
                             preferred_element_type=jnp.float32)

    return pl.pallas_call(
        body,
        grid=(8,),
        in_specs=[pl.BlockSpec((RB, D), lambda i: (i, 0)),
                  pl.BlockSpec((D, Hp), lambda i: (0, 0))],
        out_specs=pl.BlockSpec((RB, Hp), lambda i: (i, 0)),
        out_shape=jax.ShapeDtypeStruct((NP, Hp), jnp.float32),
    )(xp, W1p)


def _tc_scale(h1, degp):
    NP, Hp = h1.shape
    RB = NP // 8

    def body(deg_ref, h1_ref, h1s_ref, dinv_ref):
        deg = deg_ref[0] + deg_ref[1] + 1.0
        dinv = lax.rsqrt(jnp.maximum(deg, 1.0))
        dinv_ref[...] = dinv
        h1s_ref[...] = h1_ref[...] * dinv[:, 0:1]

    return pl.pallas_call(
        body,
        grid=(8,),
        in_specs=[pl.BlockSpec((NC, RB, L), lambda i: (0, i, 0)),
                  pl.BlockSpec((RB, Hp), lambda i: (i, 0))],
        out_specs=[pl.BlockSpec((RB, Hp), lambda i: (i, 0)),
                   pl.BlockSpec((RB, L), lambda i: (i, 0))],
        out_shape=[jax.ShapeDtypeStruct((NP, Hp), jnp.float32),
                   jax.ShapeDtypeStruct((NP, L), jnp.float32)],
    )(degp, h1)


def _tc_layer2(p, h1s, dinv, b1p, W2p):
    NP, Hp = h1s.shape
    Cp = W2p.shape[1]
    RB = NP // 8

    def body(p_ref, h1s_ref, dinv_ref, b1_ref, w2_ref, gs_ref):
        d = dinv_ref[:, 0:1]
        h = jnp.maximum((p_ref[0] + p_ref[1] + h1s_ref[...]) * d + b1_ref[...],
                        0.0)
        gs_ref[...] = jnp.dot(h, w2_ref[...],
                              preferred_element_type=jnp.float32) * d

    return pl.pallas_call(
        body,
        grid=(8,),
        in_specs=[pl.BlockSpec((NC, RB, Hp), lambda i: (0, i, 0)),
                  pl.BlockSpec((RB, Hp), lambda i: (i, 0)),
                  pl.BlockSpec((RB, L), lambda i: (i, 0)),
                  pl.BlockSpec((1, Hp), lambda i: (0, 0)),
                  pl.BlockSpec((Hp, Cp), lambda i: (0, 0))],
        out_specs=pl.BlockSpec((RB, Cp), lambda i: (i, 0)),
        out_shape=jax.ShapeDtypeStruct((NP, Cp), jnp.float32),
    )(p, h1s, dinv, b1p, W2p)


def _tc_final(q, gs, dinv, b2p, Wcp, bcp, N, C):
    NP, Cp = gs.shape
    RB = None
    for g in (5, 4, 8, 10, 16, 20, 25):
        if N % g == 0 and (N // g) % 8 == 0:
            RB = N // g
            ngrid = g
            break

    def body(q_ref, gs_ref, dinv_ref, b2_ref, wc_ref, bc_ref, h2_ref, out_ref):
        d = dinv_ref[:, 0:1]
        h2 = (q_ref[0] + q_ref[1] + gs_ref[...]) * d + b2_ref[...]
        outp = jnp.dot(h2, wc_ref[...],
                       preferred_element_type=jnp.float32) + bc_ref[...]
        if RB is not None:
            h2_ref[...] = h2[:, :C]
            out_ref[...] = outp[:, :C]
        else:
            h2_ref[...] = h2
            out_ref[...] = outp

    if RB is None:
        RB = NP // 8
        h2p, outp = pl.pallas_call(
            body,
            grid=(8,),
            in_specs=[pl.BlockSpec((NC, RB, Cp), lambda i: (0, i, 0)),
                      pl.BlockSpec((RB, Cp), lambda i: (i, 0)),
                      pl.BlockSpec((RB, L), lambda i: (i, 0)),
                      pl.BlockSpec((1, Cp), lambda i: (0, 0)),
                      pl.BlockSpec((Cp, Cp), lambda i: (0, 0)),
                      pl.BlockSpec((1, Cp), lambda i: (0, 0))],
            out_specs=[pl.BlockSpec((RB, Cp), lambda i: (i, 0)),
                       pl.BlockSpec((RB, Cp), lambda i: (i, 0))],
            out_shape=[jax.ShapeDtypeStruct((NP, Cp), jnp.float32),
                       jax.ShapeDtypeStruct((NP, Cp), jnp.float32)],
        )(q, gs, dinv, b2p, Wcp, bcp)
        return h2p[:N, :C], outp[:N, :C]

    h2, out = pl.pallas_call(
        body,
        grid=(ngrid,),
        in_specs=[pl.BlockSpec((NC, RB, Cp), lambda i: (0, i, 0)),
                  pl.BlockSpec((RB, Cp), lambda i: (i, 0)),
                  pl.BlockSpec((RB, L), lambda i: (i, 0)),
                  pl.BlockSpec((1, Cp), lambda i: (0, 0)),
                  pl.BlockSpec((Cp, Cp), lambda i: (0, 0)),
                  pl.BlockSpec((1, Cp), lambda i: (0, 0))],
        out_specs=[pl.BlockSpec((RB, C), lambda i: (i, 0)),
                   pl.BlockSpec((RB, C), lambda i: (i, 0))],
        out_shape=[jax.ShapeDtypeStruct((N, C), jnp.float32),
                   jax.ShapeDtypeStruct((N, C), jnp.float32)],
    )(q, gs, dinv, b2p, Wcp, bcp)
    return h2, out


def kernel(x, edge_index, W1, b1, W2, b2, Wc, bc):
    N, D = x.shape
    H = W1.shape[1]
    C = W2.shape[1]
    E = edge_index.shape[1]
    NW = NC * NS
    NP = _ceil(N + 1, 128) * 128
    Hp = _ceil(H, L) * L
    Cp = L

    NW8 = NW * 8
    Epad = _ceil(E, NW8) * NW8
    ei = edge_index.astype(jnp.int32)
    if Epad != E:
        padv = N + (jnp.arange(Epad - E, dtype=jnp.int32) % (NP - N))
        ei = jnp.concatenate([ei, jnp.stack([padv, padv])], axis=1)
    EPW = Epad // NW

    xp = jnp.pad(x, ((0, NP - N), (0, 0)))
    W1p = jnp.pad(W1, ((0, 0), (0, Hp - H)))
    b1p = jnp.pad(b1, (0, Hp - H))[None, :]
    W2p = jnp.pad(W2, ((0, Hp - H), (0, Cp - C)))
    b2p = jnp.pad(b2, (0, Cp - C))[None, :]
    Wcp = jnp.pad(Wc, ((0, Cp - C), (0, Cp - C)))
    bcp = jnp.pad(bc, (0, Cp - C))[None, :]

    degp = _sc_degree(ei, NP, EPW)
    h1 = _tc_matmul(xp, W1p)
    h1s, dinv = _tc_scale(h1, degp)
    p = _sc_propagate(h1s, ei, EPW)
    gs = _tc_layer2(p, h1s, dinv, b1p, W2p)
    q = _sc_propagate(gs, ei, EPW)
    h2, out = _tc_final(q, gs, dinv, b2p, Wcp, bcp, N, C)
    return (out, h2)


# speedup vs baseline: 1.0220x; 1.0117x over previous
import functools

import jax
import jax.numpy as jnp
from jax import lax
from jax.experimental import pallas as pl
from jax.experimental.pallas import tpu as pltpu
from jax.experimental.pallas import tpu_sc as plsc

NC = 2
NS = 16
L = 16
EC = 320


def _ceil(a, b):
    return -(-a // b)


def _sc_degree(ei, NP, EPW):
    stripe = NP // NS
    KF = EPW // EC
    TR = EPW - KF * EC
    mesh = plsc.VectorSubcoreMesh(core_axis_name="c", subcore_axis_name="s")

    @functools.partial(
        pl.kernel,
        out_type=jax.ShapeDtypeStruct((NC, NP, L), jnp.float32),
        mesh=mesh,
        compiler_params=pltpu.CompilerParams(use_tc_tiling_on_sc=False),
        scratch_types=[
            pltpu.VMEM((EPW,), jnp.int32),
            pltpu.VMEM((EC, L), jnp.float32),
            pltpu.VMEM((EC, L), jnp.float32),
            pltpu.VMEM_SHARED((NP, L), jnp.float32),
            pltpu.SemaphoreType.DMA,
        ],
    )
    def k(ei_hbm, out_hbm, dst_v, zero_v, one_v, acc, dsem):
        c = lax.axis_index("c")
        s = lax.axis_index("s")
        w = c * NS + s

        def fill(i, _):
            zero_v[i, :] = jnp.zeros((L,), jnp.float32)
            one_v[i, :] = jnp.ones((L,), jnp.float32)
            return _

        lax.fori_loop(0, EC, fill, 0)
        tb = s * stripe
        for b in range(stripe // EC):
            pltpu.sync_copy(zero_v, acc.at[pl.ds(tb + b * EC, EC)])
        rem = stripe - (stripe // EC) * EC
        if rem:
            pltpu.sync_copy(zero_v.at[pl.ds(0, rem)],
                            acc.at[pl.ds(tb + (stripe // EC) * EC, rem)])
        pltpu.sync_copy(ei_hbm.at[1, pl.ds(w * EPW, EPW)], dst_v)
        plsc.subcore_barrier()

        def chunk(j, _):
            pltpu.async_copy(one_v, acc.at[dst_v.at[pl.ds(j * EC, EC)]],
                             dsem, add=True)
            return _

        lax.fori_loop(0, KF, chunk, 0)
        if TR:
            pltpu.sync_copy(one_v.at[pl.ds(0, TR)],
                            acc.at[dst_v.at[pl.ds(KF * EC, TR)]], add=True)

        def drain(j, _):
            pltpu.make_async_copy(one_v, acc.at[dst_v.at[pl.ds(j * EC, EC)]],
                                  dsem).wait()
            return _

        lax.fori_loop(0, KF, drain, 0)
        plsc.subcore_barrier()
        pltpu.sync_copy(acc.at[pl.ds(tb, stripe)],
                        out_hbm.at[c, pl.ds(tb, stripe)])

    return k(ei)


def _sc_propagate(table, ei, EPW):
    NP, D = table.shape
    stripe = NP // NS
    KF = EPW // EC
    TR = EPW - KF * EC
    KP = KF // 2
    mesh = plsc.VectorSubcoreMesh(core_axis_name="c", subcore_axis_name="s")

    @functools.partial(
        pl.kernel,
        out_type=jax.ShapeDtypeStruct((NC, NP, D), jnp.float32),
        mesh=mesh,
        compiler_params=pltpu.CompilerParams(use_tc_tiling_on_sc=False),
        scratch_types=[
            pltpu.VMEM((EPW,), jnp.int32),
            pltpu.VMEM((EPW,), jnp.int32),
            pltpu.VMEM((EC, D), jnp.float32),
            pltpu.VMEM((EC, D), jnp.float32),
            pltpu.VMEM_SHARED((NP, D), jnp.float32),
            pltpu.SemaphoreType.DMA,
            pltpu.SemaphoreType.DMA,
            pltpu.SemaphoreType.DMA,
            pltpu.SemaphoreType.DMA,
        ],
    )
    def k(table_hbm, ei_hbm, out_hbm, src_v, dst_v, r0, r1,
          acc, g0, g1, s0, s1):
        rows = [r0, r1]
        gsem = [g0, g1]
        ssem = [s0, s1]
        c = lax.axis_index("c")
        s = lax.axis_index("s")
        w = c * NS + s

        def zrow(i, _):
            for t in range(D // L):
                r0[i, pl.ds(t * L, L)] = jnp.zeros((L,), jnp.float32)
            return _

        lax.fori_loop(0, EC, zrow, 0)
        tb = s * stripe
        for b in range(stripe // EC):
            pltpu.sync_copy(r0, acc.at[pl.ds(tb + b * EC, EC)])
        rem = stripe - (stripe // EC) * EC
        if rem:
            pltpu.sync_copy(r0.at[pl.ds(0, rem)],
                            acc.at[pl.ds(tb + (stripe // EC) * EC, rem)])
        pltpu.sync_copy(ei_hbm.at[0, pl.ds(w * EPW, EPW)], src_v)
        pltpu.sync_copy(ei_hbm.at[1, pl.ds(w * EPW, EPW)], dst_v)
        plsc.subcore_barrier()

        def sidx(j):
            return src_v.at[pl.ds(j * EC, EC)]

        def didx(j):
            return dst_v.at[pl.ds(j * EC, EC)]

        def gwait(slot, j):
            pltpu.make_async_copy(table_hbm.at[sidx(j)], rows[slot],
                                  gsem[slot]).wait()

        def swait(slot, j):
            pltpu.make_async_copy(rows[slot], acc.at[didx(j)],
                                  ssem[slot]).wait()

        if KP:
            pltpu.async_copy(table_hbm.at[sidx(0)], rows[0], gsem[0])
            pltpu.async_copy(table_hbm.at[sidx(1)], rows[1], gsem[1])

            def chunk2(jj, _):
                j0 = 2 * jj
                for i in range(2):
                    j = j0 + i
                    gwait(i, j)
                    pltpu.async_copy(rows[i], acc.at[didx(j)],
                                     ssem[i], add=True)
                for i in range(2):
                    j = j0 + i
                    swait(i, j)

                    @pl.when(jj < KP - 1)
                    def _ig():
                        pltpu.async_copy(table_hbm.at[sidx(j + 2)],
                                         rows[i], gsem[i])
                return _

            lax.fori_loop(0, KP, chunk2, 0)

        extras = []
        if KF % 2:
            extras.append((2 * KP * EC, EC))
        if TR:
            extras.append((KF * EC, TR))
        for off, sz in extras:
            si = src_v.at[pl.ds(off, sz)]
            di = dst_v.at[pl.ds(off, sz)]
            rs = r0.at[pl.ds(0, sz)]
            pltpu.async_copy(table_hbm.at[si], rs, g0).wait()
            pltpu.sync_copy(rs, acc.at[di], add=True)
        plsc.subcore_barrier()
        pltpu.sync_copy(acc.at[pl.ds(tb, stripe)],
                        out_hbm.at[c, pl.ds(tb, stripe)])

    return k(table, ei)


def _tc_matmul(xp, W1p):
    NP, D = xp.shape
    Hp = W1p.shape[1]
    RB = NP // 8

    def body(x_ref, w_ref, o_ref):
        o_ref[...] = jnp.dot(x_ref[...], w_ref[...],
                             preferred_element_type=jnp.float32)

    return pl.pallas_call(
        body,
        grid=(8,),
        in_specs=[pl.BlockSpec((RB, D), lambda i: (i, 0)),
                  pl.BlockSpec((D, Hp), lambda i: (0, 0))],
        out_specs=pl.BlockSpec((RB, Hp), lambda i: (i, 0)),
        out_shape=jax.ShapeDtypeStruct((NP, Hp), jnp.float32),
    )(xp, W1p)


def _tc_scale(h1, degp):
    NP, Hp = h1.shape
    RB = NP // 8

    def body(deg_ref, h1_ref, h1s_ref, dinv_ref):
        deg = deg_ref[0] + deg_ref[1] + 1.0
        dinv = lax.rsqrt(jnp.maximum(deg, 1.0))
        dinv_ref[...] = dinv
        h1s_ref[...] = h1_ref[...] * dinv[:, 0:1]

    return pl.pallas_call(
        body,
        grid=(8,),
        in_specs=[pl.BlockSpec((NC, RB, L), lambda i: (0, i, 0)),
                  pl.BlockSpec((RB, Hp), lambda i: (i, 0))],
        out_specs=[pl.BlockSpec((RB, Hp), lambda i: (i, 0)),
                   pl.BlockSpec((RB, L), lambda i: (i, 0))],
        out_shape=[jax.ShapeDtypeStruct((NP, Hp), jnp.float32),
                   jax.ShapeDtypeStruct((NP, L), jnp.float32)],
    )(degp, h1)


def _tc_layer2(p, h1s, dinv, b1p, W2p):
    NP, Hp = h1s.shape
    Cp = W2p.shape[1]
    RB = NP // 8

    def body(p_ref, h1s_ref, dinv_ref, b1_ref, w2_ref, gs_ref):
        d = dinv_ref[:, 0:1]
        h = jnp.maximum((p_ref[0] + p_ref[1] + h1s_ref[...]) * d + b1_ref[...],
                        0.0)
        gs_ref[...] = jnp.dot(h, w2_ref[...],
                              preferred_element_type=jnp.float32) * d

    return pl.pallas_call(
        body,
        grid=(8,),
        in_specs=[pl.BlockSpec((NC, RB, Hp), lambda i: (0, i, 0)),
                  pl.BlockSpec((RB, Hp), lambda i: (i, 0)),
                  pl.BlockSpec((RB, L), lambda i: (i, 0)),
                  pl.BlockSpec((1, Hp), lambda i: (0, 0)),
                  pl.BlockSpec((Hp, Cp), lambda i: (0, 0))],
        out_specs=pl.BlockSpec((RB, Cp), lambda i: (i, 0)),
        out_shape=jax.ShapeDtypeStruct((NP, Cp), jnp.float32),
    )(p, h1s, dinv, b1p, W2p)


def _tc_final(q, gs, dinv, b2p, Wcp, bcp, N, C):
    NP, Cp = gs.shape
    RB = None
    for g in (5, 4, 8, 10, 16, 20, 25):
        if N % g == 0 and (N // g) % 8 == 0:
            RB = N // g
            ngrid = g
            break

    def body(q_ref, gs_ref, dinv_ref, b2_ref, wc_ref, bc_ref, h2_ref, out_ref):
        d = dinv_ref[:, 0:1]
        h2 = (q_ref[0] + q_ref[1] + gs_ref[...]) * d + b2_ref[...]
        outp = jnp.dot(h2, wc_ref[...],
                       preferred_element_type=jnp.float32) + bc_ref[...]
        if RB is not None:
            h2_ref[...] = h2[:, :C]
            out_ref[...] = outp[:, :C]
        else:
            h2_ref[...] = h2
            out_ref[...] = outp

    if RB is None:
        RB = NP // 8
        h2p, outp = pl.pallas_call(
            body,
            grid=(8,),
            in_specs=[pl.BlockSpec((NC, RB, Cp), lambda i: (0, i, 0)),
                      pl.BlockSpec((RB, Cp), lambda i: (i, 0)),
                      pl.BlockSpec((RB, L), lambda i: (i, 0)),
                      pl.BlockSpec((1, Cp), lambda i: (0, 0)),
                      pl.BlockSpec((Cp, Cp), lambda i: (0, 0)),
                      pl.BlockSpec((1, Cp), lambda i: (0, 0))],
            out_specs=[pl.BlockSpec((RB, Cp), lambda i: (i, 0)),
                       pl.BlockSpec((RB, Cp), lambda i: (i, 0))],
            out_shape=[jax.ShapeDtypeStruct((NP, Cp), jnp.float32),
                       jax.ShapeDtypeStruct((NP, Cp), jnp.float32)],
        )(q, gs, dinv, b2p, Wcp, bcp)
        return h2p[:N, :C], outp[:N, :C]

    h2, out = pl.pallas_call(
        body,
        grid=(ngrid,),
        in_specs=[pl.BlockSpec((NC, RB, Cp), lambda i: (0, i, 0)),
                  pl.BlockSpec((RB, Cp), lambda i: (i, 0)),
                  pl.BlockSpec((RB, L), lambda i: (i, 0)),
                  pl.BlockSpec((1, Cp), lambda i: (0, 0)),
                  pl.BlockSpec((Cp, Cp), lambda i: (0, 0)),
                  pl.BlockSpec((1, Cp), lambda i: (0, 0))],
        out_specs=[pl.BlockSpec((RB, C), lambda i: (i, 0)),
                   pl.BlockSpec((RB, C), lambda i: (i, 0))],
        out_shape=[jax.ShapeDtypeStruct((N, C), jnp.float32),
                   jax.ShapeDtypeStruct((N, C), jnp.float32)],
    )(q, gs, dinv, b2p, Wcp, bcp)
    return h2, out


def kernel(x, edge_index, W1, b1, W2, b2, Wc, bc):
    N, D = x.shape
    H = W1.shape[1]
    C = W2.shape[1]
    E = edge_index.shape[1]
    NW = NC * NS
    NP = _ceil(N + 1, 128) * 128
    Hp = _ceil(H, L) * L
    Cp = L

    NW8 = NW * 8
    Epad = _ceil(E, NW8) * NW8
    ei = edge_index.astype(jnp.int32)
    if Epad != E:
        padv = N + (jnp.arange(Epad - E, dtype=jnp.int32) % (NP - N))
        ei = jnp.concatenate([ei, jnp.stack([padv, padv])], axis=1)
    EPW = Epad // NW

    xp = jnp.pad(x, ((0, NP - N), (0, 0)))
    W1p = jnp.pad(W1, ((0, 0), (0, Hp - H)))
    b1p = jnp.pad(b1, (0, Hp - H))[None, :]
    W2p = jnp.pad(W2, ((0, Hp - H), (0, Cp - C)))
    b2p = jnp.pad(b2, (0, Cp - C))[None, :]
    Wcp = jnp.pad(Wc, ((0, Cp - C), (0, Cp - C)))
    bcp = jnp.pad(bc, (0, Cp - C))[None, :]

    degp = _sc_degree(ei, NP, EPW)
    h1 = _tc_matmul(xp, W1p)
    h1s, dinv = _tc_scale(h1, degp)
    p = _sc_propagate(h1s, ei, EPW)
    gs = _tc_layer2(p, h1s, dinv, b1p, W2p)
    q = _sc_propagate(gs, ei, EPW)
    h2, out = _tc_final(q, gs, dinv, b2p, Wcp, bcp, N, C)
    return (out, h2)


# speedup vs baseline: 1.0272x; 1.0051x over previous
import functools

import jax
import jax.numpy as jnp
from jax import lax
from jax.experimental import pallas as pl
from jax.experimental.pallas import tpu as pltpu
from jax.experimental.pallas import tpu_sc as plsc

NC = 2
NS = 16
L = 16
EC = 360


def _ceil(a, b):
    return -(-a // b)


def _sc_degree(ei, NP, EPW):
    stripe = NP // NS
    KF = EPW // EC
    TR = EPW - KF * EC
    mesh = plsc.VectorSubcoreMesh(core_axis_name="c", subcore_axis_name="s")

    @functools.partial(
        pl.kernel,
        out_type=jax.ShapeDtypeStruct((NC, NP, L), jnp.float32),
        mesh=mesh,
        compiler_params=pltpu.CompilerParams(use_tc_tiling_on_sc=False),
        scratch_types=[
            pltpu.VMEM((EPW,), jnp.int32),
            pltpu.VMEM((EC, L), jnp.float32),
            pltpu.VMEM((EC, L), jnp.float32),
            pltpu.VMEM_SHARED((NP, L), jnp.float32),
            pltpu.SemaphoreType.DMA,
        ],
    )
    def k(ei_hbm, out_hbm, dst_v, zero_v, one_v, acc, dsem):
        c = lax.axis_index("c")
        s = lax.axis_index("s")
        w = c * NS + s

        def fill(i, _):
            zero_v[i, :] = jnp.zeros((L,), jnp.float32)
            one_v[i, :] = jnp.ones((L,), jnp.float32)
            return _

        lax.fori_loop(0, EC, fill, 0)
        tb = s * stripe
        for b in range(stripe // EC):
            pltpu.sync_copy(zero_v, acc.at[pl.ds(tb + b * EC, EC)])
        rem = stripe - (stripe // EC) * EC
        if rem:
            pltpu.sync_copy(zero_v.at[pl.ds(0, rem)],
                            acc.at[pl.ds(tb + (stripe // EC) * EC, rem)])
        pltpu.sync_copy(ei_hbm.at[1, pl.ds(w * EPW, EPW)], dst_v)
        plsc.subcore_barrier()

        def chunk(j, _):
            pltpu.async_copy(one_v, acc.at[dst_v.at[pl.ds(j * EC, EC)]],
                             dsem, add=True)
            return _

        lax.fori_loop(0, KF, chunk, 0)
        if TR:
            pltpu.sync_copy(one_v.at[pl.ds(0, TR)],
                            acc.at[dst_v.at[pl.ds(KF * EC, TR)]], add=True)

        def drain(j, _):
            pltpu.make_async_copy(one_v, acc.at[dst_v.at[pl.ds(j * EC, EC)]],
                                  dsem).wait()
            return _

        lax.fori_loop(0, KF, drain, 0)
        plsc.subcore_barrier()
        pltpu.sync_copy(acc.at[pl.ds(tb, stripe)],
                        out_hbm.at[c, pl.ds(tb, stripe)])

    return k(ei)


def _sc_propagate(table, ei, EPW):
    NP, D = table.shape
    stripe = NP // NS
    KF = EPW // EC
    TR = EPW - KF * EC
    KP = KF // 2
    mesh = plsc.VectorSubcoreMesh(core_axis_name="c", subcore_axis_name="s")

    @functools.partial(
        pl.kernel,
        out_type=jax.ShapeDtypeStruct((NC, NP, D), jnp.float32),
        mesh=mesh,
        compiler_params=pltpu.CompilerParams(use_tc_tiling_on_sc=False),
        scratch_types=[
            pltpu.VMEM((EPW,), jnp.int32),
            pltpu.VMEM((EPW,), jnp.int32),
            pltpu.VMEM((EC, D), jnp.float32),
            pltpu.VMEM((EC, D), jnp.float32),
            pltpu.VMEM_SHARED((NP, D), jnp.float32),
            pltpu.SemaphoreType.DMA,
            pltpu.SemaphoreType.DMA,
            pltpu.SemaphoreType.DMA,
            pltpu.SemaphoreType.DMA,
        ],
    )
    def k(table_hbm, ei_hbm, out_hbm, src_v, dst_v, r0, r1,
          acc, g0, g1, s0, s1):
        rows = [r0, r1]
        gsem = [g0, g1]
        ssem = [s0, s1]
        c = lax.axis_index("c")
        s = lax.axis_index("s")
        w = c * NS + s

        def zrow(i, _):
            for t in range(D // L):
                r0[i, pl.ds(t * L, L)] = jnp.zeros((L,), jnp.float32)
            return _

        lax.fori_loop(0, EC, zrow, 0)
        tb = s * stripe
        for b in range(stripe // EC):
            pltpu.sync_copy(r0, acc.at[pl.ds(tb + b * EC, EC)])
        rem = stripe - (stripe // EC) * EC
        if rem:
            pltpu.sync_copy(r0.at[pl.ds(0, rem)],
                            acc.at[pl.ds(tb + (stripe // EC) * EC, rem)])
        pltpu.sync_copy(ei_hbm.at[0, pl.ds(w * EPW, EPW)], src_v)
        pltpu.sync_copy(ei_hbm.at[1, pl.ds(w * EPW, EPW)], dst_v)
        plsc.subcore_barrier()

        def sidx(j):
            return src_v.at[pl.ds(j * EC, EC)]

        def didx(j):
            return dst_v.at[pl.ds(j * EC, EC)]

        def gwait(slot, j):
            pltpu.make_async_copy(table_hbm.at[sidx(j)], rows[slot],
                                  gsem[slot]).wait()

        def swait(slot, j):
            pltpu.make_async_copy(rows[slot], acc.at[didx(j)],
                                  ssem[slot]).wait()

        if KP:
            pltpu.async_copy(table_hbm.at[sidx(0)], rows[0], gsem[0])
            pltpu.async_copy(table_hbm.at[sidx(1)], rows[1], gsem[1])

            def chunk2(jj, _):
                j0 = 2 * jj
                for i in range(2):
                    j = j0 + i
                    gwait(i, j)
                    pltpu.async_copy(rows[i], acc.at[didx(j)],
                                     ssem[i], add=True)
                for i in range(2):
                    j = j0 + i
                    swait(i, j)

                    @pl.when(jj < KP - 1)
                    def _ig():
                        pltpu.async_copy(table_hbm.at[sidx(j + 2)],
                                         rows[i], gsem[i])
                return _

            lax.fori_loop(0, KP, chunk2, 0)

        extras = []
        if KF % 2:
            extras.append((2 * KP * EC, EC))
        if TR:
            extras.append((KF * EC, TR))
        for off, sz in extras:
            si = src_v.at[pl.ds(off, sz)]
            di = dst_v.at[pl.ds(off, sz)]
            rs = r0.at[pl.ds(0, sz)]
            pltpu.async_copy(table_hbm.at[si], rs, g0).wait()
            pltpu.sync_copy(rs, acc.at[di], add=True)
        plsc.subcore_barrier()
        pltpu.sync_copy(acc.at[pl.ds(tb, stripe)],
                        out_hbm.at[c, pl.ds(tb, stripe)])

    return k(table, ei)


def _tc_matmul(xp, W1p):
    NP, D = xp.shape
    Hp = W1p.shape[1]
    RB = NP // 8

    def body(x_ref, w_ref, o_ref):
        o_ref[...] = jnp.dot(x_ref[...], w_ref[...],
                             preferred_element_type=jnp.float32)

    return pl.pallas_call(
        body,
        grid=(8,),
        in_specs=[pl.BlockSpec((RB, D), lambda i: (i, 0)),
                  pl.BlockSpec((D, Hp), lambda i: (0, 0))],
        out_specs=pl.BlockSpec((RB, Hp), lambda i: (i, 0)),
        out_shape=jax.ShapeDtypeStruct((NP, Hp), jnp.float32),
    )(xp, W1p)


def _tc_scale(h1, degp):
    NP, Hp = h1.shape
    RB = NP // 8

    def body(deg_ref, h1_ref, h1s_ref, dinv_ref):
        deg = deg_ref[0] + deg_ref[1] + 1.0
        dinv = lax.rsqrt(jnp.maximum(deg, 1.0))
        dinv_ref[...] = dinv
        h1s_ref[...] = h1_ref[...] * dinv[:, 0:1]

    return pl.pallas_call(
        body,
        grid=(8,),
        in_specs=[pl.BlockSpec((NC, RB, L), lambda i: (0, i, 0)),
                  pl.BlockSpec((RB, Hp), lambda i: (i, 0))],
        out_specs=[pl.BlockSpec((RB, Hp), lambda i: (i, 0)),
                   pl.BlockSpec((RB, L), lambda i: (i, 0))],
        out_shape=[jax.ShapeDtypeStruct((NP, Hp), jnp.float32),
                   jax.ShapeDtypeStruct((NP, L), jnp.float32)],
    )(degp, h1)


def _tc_layer2(p, h1s, dinv, b1p, W2p):
    NP, Hp = h1s.shape
    Cp = W2p.shape[1]
    RB = NP // 8

    def body(p_ref, h1s_ref, dinv_ref, b1_ref, w2_ref, gs_ref):
        d = dinv_ref[:, 0:1]
        h = jnp.maximum((p_ref[0] + p_ref[1] + h1s_ref[...]) * d + b1_ref[...],
                        0.0)
        gs_ref[...] = jnp.dot(h, w2_ref[...],
                              preferred_element_type=jnp.float32) * d

    return pl.pallas_call(
        body,
        grid=(8,),
        in_specs=[pl.BlockSpec((NC, RB, Hp), lambda i: (0, i, 0)),
                  pl.BlockSpec((RB, Hp), lambda i: (i, 0)),
                  pl.BlockSpec((RB, L), lambda i: (i, 0)),
                  pl.BlockSpec((1, Hp), lambda i: (0, 0)),
                  pl.BlockSpec((Hp, Cp), lambda i: (0, 0))],
        out_specs=pl.BlockSpec((RB, Cp), lambda i: (i, 0)),
        out_shape=jax.ShapeDtypeStruct((NP, Cp), jnp.float32),
    )(p, h1s, dinv, b1p, W2p)


def _tc_final(q, gs, dinv, b2p, Wcp, bcp, N, C):
    NP, Cp = gs.shape
    RB = None
    for g in (5, 4, 8, 10, 16, 20, 25):
        if N % g == 0 and (N // g) % 8 == 0:
            RB = N // g
            ngrid = g
            break

    def body(q_ref, gs_ref, dinv_ref, b2_ref, wc_ref, bc_ref, h2_ref, out_ref):
        d = dinv_ref[:, 0:1]
        h2 = (q_ref[0] + q_ref[1] + gs_ref[...]) * d + b2_ref[...]
        outp = jnp.dot(h2, wc_ref[...],
                       preferred_element_type=jnp.float32) + bc_ref[...]
        if RB is not None:
            h2_ref[...] = h2[:, :C]
            out_ref[...] = outp[:, :C]
        else:
            h2_ref[...] = h2
            out_ref[...] = outp

    if RB is None:
        RB = NP // 8
        h2p, outp = pl.pallas_call(
            body,
            grid=(8,),
            in_specs=[pl.BlockSpec((NC, RB, Cp), lambda i: (0, i, 0)),
                      pl.BlockSpec((RB, Cp), lambda i: (i, 0)),
                      pl.BlockSpec((RB, L), lambda i: (i, 0)),
                      pl.BlockSpec((1, Cp), lambda i: (0, 0)),
                      pl.BlockSpec((Cp, Cp), lambda i: (0, 0)),
                      pl.BlockSpec((1, Cp), lambda i: (0, 0))],
            out_specs=[pl.BlockSpec((RB, Cp), lambda i: (i, 0)),
                       pl.BlockSpec((RB, Cp), lambda i: (i, 0))],
            out_shape=[jax.ShapeDtypeStruct((NP, Cp), jnp.float32),
                       jax.ShapeDtypeStruct((NP, Cp), jnp.float32)],
        )(q, gs, dinv, b2p, Wcp, bcp)
        return h2p[:N, :C], outp[:N, :C]

    h2, out = pl.pallas_call(
        body,
        grid=(ngrid,),
        in_specs=[pl.BlockSpec((NC, RB, Cp), lambda i: (0, i, 0)),
                  pl.BlockSpec((RB, Cp), lambda i: (i, 0)),
                  pl.BlockSpec((RB, L), lambda i: (i, 0)),
                  pl.BlockSpec((1, Cp), lambda i: (0, 0)),
                  pl.BlockSpec((Cp, Cp), lambda i: (0, 0)),
                  pl.BlockSpec((1, Cp), lambda i: (0, 0))],
        out_specs=[pl.BlockSpec((RB, C), lambda i: (i, 0)),
                   pl.BlockSpec((RB, C), lambda i: (i, 0))],
        out_shape=[jax.ShapeDtypeStruct((N, C), jnp.float32),
                   jax.ShapeDtypeStruct((N, C), jnp.float32)],
    )(q, gs, dinv, b2p, Wcp, bcp)
    return h2, out


def kernel(x, edge_index, W1, b1, W2, b2, Wc, bc):
    N, D = x.shape
    H = W1.shape[1]
    C = W2.shape[1]
    E = edge_index.shape[1]
    NW = NC * NS
    NP = _ceil(N + 1, 128) * 128
    Hp = _ceil(H, L) * L
    Cp = L

    NW8 = NW * 8
    Epad = _ceil(E, NW8) * NW8
    ei = edge_index.astype(jnp.int32)
    if Epad != E:
        padv = N + (jnp.arange(Epad - E, dtype=jnp.int32) % (NP - N))
        ei = jnp.concatenate([ei, jnp.stack([padv, padv])], axis=1)
    EPW = Epad // NW

    xp = jnp.pad(x, ((0, NP - N), (0, 0)))
    W1p = jnp.pad(W1, ((0, 0), (0, Hp - H)))
    b1p = jnp.pad(b1, (0, Hp - H))[None, :]
    W2p = jnp.pad(W2, ((0, Hp - H), (0, Cp - C)))
    b2p = jnp.pad(b2, (0, Cp - C))[None, :]
    Wcp = jnp.pad(Wc, ((0, Cp - C), (0, Cp - C)))
    bcp = jnp.pad(bc, (0, Cp - C))[None, :]

    degp = _sc_degree(ei, NP, EPW)
    h1 = _tc_matmul(xp, W1p)
    h1s, dinv = _tc_scale(h1, degp)
    p = _sc_propagate(h1s, ei, EPW)
    gs = _tc_layer2(p, h1s, dinv, b1p, W2p)
    q = _sc_propagate(gs, ei, EPW)
    h2, out = _tc_final(q, gs, dinv, b2p, Wcp, bcp, N, C)
    return (out, h2)
